# Initial kernel scaffold; baseline (speedup 1.0000x reference)
import jax
import jax.numpy as jnp
from jax.experimental import pallas as pl


def kernel(mh_output, W_route, b_route, W_noise, b_noise, noise_eps):
    raise NotImplementedError("write your pallas kernel here")


# speedup vs baseline: 1.0400x; 1.0400x over previous
import functools

import jax
import jax.numpy as jnp
from jax.experimental import pallas as pl

N_TOK = 16384
N_EMBD = 2048
N_EXPERTS = 16
TOP_K = 2

TILE = 1024


def _router_krn(x_ref, wr_ref, br_ref, wn_ref, bn_ref, eps_ref,
                rout_ref, idx_ref, g1_ref):
    x = x_ref[...]
    logits = jnp.dot(x, wr_ref[...], preferred_element_type=jnp.float32) + br_ref[...]
    nlog = jnp.dot(x, wn_ref[...], preferred_element_type=jnp.float32) + bn_ref[...]
    noisy = logits + eps_ref[...] * jax.nn.softplus(nlog)

    m1 = jnp.max(noisy, axis=-1, keepdims=True)
    e_all = jnp.exp(noisy - m1)
    g1_ref[...] = e_all / jnp.sum(e_all, axis=-1, keepdims=True)

    lane = jax.lax.broadcasted_iota(jnp.int32, noisy.shape, 1)
    big = jnp.int32(N_EXPERTS)
    i1 = jnp.min(jnp.where(noisy == m1, lane, big), axis=-1, keepdims=True)
    rest = jnp.where(lane == i1, -jnp.inf, noisy)
    m2 = jnp.max(rest, axis=-1, keepdims=True)
    i2 = jnp.min(jnp.where(rest == m2, lane, big), axis=-1, keepdims=True)
    idx_ref[...] = jnp.concatenate([i1, i2], axis=-1)

    keep = (lane == i1) | (lane == i2)
    e_top = jnp.where(keep, e_all, 0.0)
    rout_ref[...] = e_top / jnp.sum(e_top, axis=-1, keepdims=True)


@functools.partial(jax.jit, static_argnames=())
def kernel(mh_output, W_route, b_route, W_noise, b_noise, noise_eps):
    grid = (N_TOK // TILE,)
    br = b_route.reshape(1, N_EXPERTS)
    bn = b_noise.reshape(1, N_EXPERTS)
    router_output, indices, gate1 = pl.pallas_call(
        _router_krn,
        grid=grid,
        in_specs=[
            pl.BlockSpec((TILE, N_EMBD), lambda i: (i, 0)),
            pl.BlockSpec((N_EMBD, N_EXPERTS), lambda i: (0, 0)),
            pl.BlockSpec((1, N_EXPERTS), lambda i: (0, 0)),
            pl.BlockSpec((N_EMBD, N_EXPERTS), lambda i: (0, 0)),
            pl.BlockSpec((1, N_EXPERTS), lambda i: (0, 0)),
            pl.BlockSpec((TILE, N_EXPERTS), lambda i: (i, 0)),
        ],
        out_specs=[
            pl.BlockSpec((TILE, N_EXPERTS), lambda i: (i, 0)),
            pl.BlockSpec((TILE, TOP_K), lambda i: (i, 0)),
            pl.BlockSpec((TILE, N_EXPERTS), lambda i: (i, 0)),
        ],
        out_shape=[
            jax.ShapeDtypeStruct((N_TOK, N_EXPERTS), jnp.float32),
            jax.ShapeDtypeStruct((N_TOK, TOP_K), jnp.int32),
            jax.ShapeDtypeStruct((N_TOK, N_EXPERTS), jnp.float32),
        ],
    )(mh_output, W_route, br, W_noise, bn, noise_eps)
    return (router_output, indices, gate1)


# speedup vs baseline: 1.1096x; 1.0670x over previous
import functools

import jax
import jax.numpy as jnp
from jax.experimental import pallas as pl

N_TOK = 16384
N_EMBD = 2048
N_EXPERTS = 16
TOP_K = 2

TILE = 1024


def _router_krn(x_ref, w_ref, b_ref, eps_ref, rout_ref, idx_ref, g1_ref):
    x = x_ref[...]
    y = jnp.dot(x, w_ref[...], preferred_element_type=jnp.float32) + b_ref[...]
    logits = y[:, :N_EXPERTS]
    nlog = y[:, N_EXPERTS:]
    noisy = logits + eps_ref[...] * jax.nn.softplus(nlog)

    m1 = jnp.max(noisy, axis=-1, keepdims=True)
    e_all = jnp.exp(noisy - m1)
    g1_ref[...] = e_all / jnp.sum(e_all, axis=-1, keepdims=True)

    lane = jax.lax.broadcasted_iota(jnp.int32, noisy.shape, 1)
    big = jnp.int32(N_EXPERTS)
    i1 = jnp.min(jnp.where(noisy == m1, lane, big), axis=-1, keepdims=True)
    rest = jnp.where(lane == i1, -jnp.inf, noisy)
    m2 = jnp.max(rest, axis=-1, keepdims=True)
    i2 = jnp.min(jnp.where(rest == m2, lane, big), axis=-1, keepdims=True)
    idx_ref[...] = jnp.concatenate([i1, i2], axis=-1)

    keep = (lane == i1) | (lane == i2)
    rout_ref[...] = jnp.where(keep, e_all, 0.0) / (1.0 + jnp.exp(m2 - m1))


@functools.partial(jax.jit, static_argnames=())
def kernel(mh_output, W_route, b_route, W_noise, b_noise, noise_eps):
    grid = (N_TOK // TILE,)
    W = jnp.concatenate([W_route, W_noise], axis=1)
    b = jnp.concatenate([b_route, b_noise]).reshape(1, 2 * N_EXPERTS)
    router_output, indices, gate1 = pl.pallas_call(
        _router_krn,
        grid=grid,
        in_specs=[
            pl.BlockSpec((TILE, N_EMBD), lambda i: (i, 0)),
            pl.BlockSpec((N_EMBD, 2 * N_EXPERTS), lambda i: (0, 0)),
            pl.BlockSpec((1, 2 * N_EXPERTS), lambda i: (0, 0)),
            pl.BlockSpec((TILE, N_EXPERTS), lambda i: (i, 0)),
        ],
        out_specs=[
            pl.BlockSpec((TILE, N_EXPERTS), lambda i: (i, 0)),
            pl.BlockSpec((TILE, TOP_K), lambda i: (i, 0)),
            pl.BlockSpec((TILE, N_EXPERTS), lambda i: (i, 0)),
        ],
        out_shape=[
            jax.ShapeDtypeStruct((N_TOK, N_EXPERTS), jnp.float32),
            jax.ShapeDtypeStruct((N_TOK, TOP_K), jnp.int32),
            jax.ShapeDtypeStruct((N_TOK, N_EXPERTS), jnp.float32),
        ],
    )(mh_output, W, b, noise_eps)
    return (router_output, indices, gate1)
